# baseline (speedup 1.0000x reference)
import functools

import jax
import jax.numpy as jnp
from jax import lax
from jax.experimental import pallas as pl
from jax.experimental.pallas import tpu as pltpu
from jax.experimental.pallas import tpu_sc as plsc

B, N, C = 16, 2048, 256
K = 3
KM_ITERS = 10
BISECT_ITERS = 48
SC_BISECT_ITERS = 36
NV = N // 16


def _heatmap_body(x_ref, hm_ref):
    hm_ref[0, 0, :] = jnp.sum(x_ref[0], axis=-1) * (1.0 / C)


def _heatmap(x):
    return pl.pallas_call(
        _heatmap_body,
        grid=(B,),
        in_specs=[pl.BlockSpec((1, N, C), lambda b: (b, 0, 0))],
        out_specs=pl.BlockSpec((1, 1, N), lambda b: (b, 0, 0)),
        out_shape=jax.ShapeDtypeStruct((B, 1, N), jnp.float32),
    )(x)


def _order_stat(hm, k):
    lo = jnp.min(hm, axis=1, keepdims=True) - 1.0
    hi = jnp.max(hm, axis=1, keepdims=True)

    def body(_, carry):
        lo, hi = carry
        mid = 0.5 * (lo + hi)
        cnt = jnp.sum(jnp.where(hm <= mid, 1.0, 0.0), axis=1, keepdims=True)
        ge = cnt >= (k + 1)
        return jnp.where(ge, lo, mid), jnp.where(ge, mid, hi)

    lo, hi = lax.fori_loop(0, BISECT_ITERS, body, (lo, hi))
    return hi


def _cluster_body(hm_ref, lab_ref):
    hm = hm_ref[:, 0, :]
    c0 = jnp.min(hm, axis=1, keepdims=True)
    c2 = jnp.max(hm, axis=1, keepdims=True)
    c1 = 0.5 * (_order_stat(hm, N // 2 - 1) + _order_stat(hm, N // 2))
    total_s = jnp.sum(hm, axis=1, keepdims=True)

    def lloyd(_, carry):
        c0, c1, c2 = carry
        t01 = 0.5 * (c0 + c1)
        t12 = 0.5 * (c1 + c2)
        m1 = hm > t01
        m2 = hm > t12
        s0 = jnp.sum(jnp.where(m1, 0.0, hm), axis=1, keepdims=True)
        s2 = jnp.sum(jnp.where(m2, hm, 0.0), axis=1, keepdims=True)
        n0 = jnp.sum(jnp.where(m1, 0.0, 1.0), axis=1, keepdims=True)
        n2 = jnp.sum(jnp.where(m2, 1.0, 0.0), axis=1, keepdims=True)
        s1 = total_s - s0 - s2
        n1 = N - n0 - n2
        c0 = jnp.where(n0 > 0, s0 / jnp.maximum(n0, 1.0), c0)
        c1 = jnp.where(n1 > 0, s1 / jnp.maximum(n1, 1.0), c1)
        c2 = jnp.where(n2 > 0, s2 / jnp.maximum(n2, 1.0), c2)
        return c0, c1, c2

    c0, c1, c2 = lax.fori_loop(0, KM_ITERS, lloyd, (c0, c1, c2))

    t01 = 0.5 * (c0 + c1)
    t12 = 0.5 * (c1 + c2)
    lab = jnp.where(hm > t01, 1, 0) + jnp.where(hm > t12, 1, 0)

    a0 = (jnp.where(c1 > c0, 1, 0) + jnp.where(c2 > c0, 1, 0))
    a1 = (jnp.where(c0 > c1, 1, 0) + jnp.where(c2 > c1, 1, 0)
          + jnp.where(c0 == c1, 1, 0))
    a2 = (jnp.where(c0 > c2, 1, 0) + jnp.where(c1 > c2, 1, 0)
          + jnp.where(c0 == c2, 1, 0) + jnp.where(c1 == c2, 1, 0))
    adj = jnp.where(lab == 0, a0, jnp.where(lab == 1, a1, a2))
    lab_ref[:, 0, :] = adj


def _cluster_tc(hm):
    return pl.pallas_call(
        _cluster_body,
        in_specs=[pl.BlockSpec((B, 1, N), lambda: (0, 0, 0))],
        out_specs=pl.BlockSpec((B, 1, N), lambda: (0, 0, 0)),
        out_shape=jax.ShapeDtypeStruct((B, 1, N), jnp.int32),
    )(hm)


def _sc_cluster_body(hm_hbm, lab_hbm, hm_v, lab_v):
    wid = lax.axis_index("s") * 2 + lax.axis_index("c")

    @pl.when(wid < B)
    def _():
        pltpu.sync_copy(hm_hbm.at[wid], hm_v)

        def ld(i):
            return hm_v[pl.ds(i * 16, 16)]

        v0 = ld(0)

        def p1(i, carry):
            vmin, vmax, vsum = carry
            v = ld(i)
            return (jnp.minimum(vmin, v), jnp.maximum(vmax, v), vsum + v)

        z16 = jnp.zeros((16,), jnp.float32)

        def _splat(s):
            return jnp.full((16,), s, jnp.float32)

        vmin, vmax, vsum = lax.fori_loop(1, NV, p1, (v0, v0, v0), unroll=8)
        c0 = _splat(jnp.min(vmin))
        c2 = _splat(jnp.max(vmax))
        total_s = _splat(jnp.sum(vsum))

        def bis(_, carry):
            lo1, hi1, lo2, hi2 = carry
            m1 = 0.5 * (lo1 + hi1)
            m2 = 0.5 * (lo2 + hi2)

            def cntb(i, cc):
                a1, a2 = cc
                v = ld(i)
                a1 = a1 + jnp.where(v <= m1, 1.0, 0.0)
                a2 = a2 + jnp.where(v <= m2, 1.0, 0.0)
                return a1, a2

            a1, a2 = lax.fori_loop(0, NV, cntb, (z16, z16), unroll=8)
            n1 = _splat(jnp.sum(a1))
            n2 = _splat(jnp.sum(a2))
            ge1 = n1 >= float(N // 2)
            ge2 = n2 >= float(N // 2 + 1)
            return (jnp.where(ge1, lo1, m1), jnp.where(ge1, m1, hi1),
                    jnp.where(ge2, lo2, m2), jnp.where(ge2, m2, hi2))

        lo0 = c0 - 1.0
        _, q1, _, q2 = lax.fori_loop(0, SC_BISECT_ITERS, bis,
                                     (lo0, c2, lo0, c2))
        c1 = 0.5 * (q1 + q2)

        def lloyd(_, carry):
            c0_, c1_, c2_ = carry
            t01 = 0.5 * (c0_ + c1_)
            t12 = 0.5 * (c1_ + c2_)

            def acc(i, cc):
                s0, s2, n0, n2 = cc
                v = ld(i)
                g1 = v > t01
                g2 = v > t12
                return (s0 + jnp.where(g1, 0.0, v),
                        s2 + jnp.where(g2, v, 0.0),
                        n0 + jnp.where(g1, 0.0, 1.0),
                        n2 + jnp.where(g2, 1.0, 0.0))

            s0v, s2v, n0v, n2v = lax.fori_loop(0, NV, acc,
                                               (z16, z16, z16, z16), unroll=8)
            s0 = _splat(jnp.sum(s0v))
            s2 = _splat(jnp.sum(s2v))
            n0 = _splat(jnp.sum(n0v))
            n2 = _splat(jnp.sum(n2v))
            s1 = total_s - s0 - s2
            n1 = float(N) - n0 - n2
            c0_ = jnp.where(n0 > 0, s0 / jnp.maximum(n0, 1.0), c0_)
            c1_ = jnp.where(n1 > 0, s1 / jnp.maximum(n1, 1.0), c1_)
            c2_ = jnp.where(n2 > 0, s2 / jnp.maximum(n2, 1.0), c2_)
            return c0_, c1_, c2_

        c0, c1, c2 = lax.fori_loop(0, KM_ITERS, lloyd, (c0, c1, c2))

        t01 = 0.5 * (c0 + c1)
        t12 = 0.5 * (c1 + c2)
        a0 = jnp.where(c1 > c0, 1, 0) + jnp.where(c2 > c0, 1, 0)
        a1 = (jnp.where(c0 > c1, 1, 0) + jnp.where(c2 > c1, 1, 0)
              + jnp.where(c0 == c1, 1, 0))
        a2 = (jnp.where(c0 > c2, 1, 0) + jnp.where(c1 > c2, 1, 0)
              + jnp.where(c0 == c2, 1, 0) + jnp.where(c1 == c2, 1, 0))

        def wr(i, carry):
            v = ld(i)
            lab = jnp.where(v > t01, 1, 0) + jnp.where(v > t12, 1, 0)
            adj = jnp.where(lab == 0, a0, jnp.where(lab == 1, a1, a2))
            lab_v[pl.ds(i * 16, 16)] = adj
            return carry

        lax.fori_loop(0, NV, wr, 0, unroll=8)
        pltpu.sync_copy(lab_v, lab_hbm.at[wid])


def _cluster_sc(hm):
    mesh = plsc.VectorSubcoreMesh(core_axis_name="c", subcore_axis_name="s",
                                  num_cores=2, num_subcores=16)
    f = pl.kernel(
        _sc_cluster_body,
        out_type=jax.ShapeDtypeStruct((B, N), jnp.int32),
        mesh=mesh,
        compiler_params=pltpu.CompilerParams(needs_layout_passes=False),
        scratch_types=[pltpu.VMEM((N,), jnp.float32),
                       pltpu.VMEM((N,), jnp.int32)],
    )
    return f(hm)


def _pool_body(x_ref, lab_ref, out_ref):
    lab = lab_ref[0, 0, :]
    oh = jnp.where(lab[None, :] == jax.lax.broadcasted_iota(jnp.int32, (K, N), 0),
                   1.0, 0.0)
    sums = jax.lax.dot(oh, x_ref[0], precision=jax.lax.Precision.HIGHEST,
                       preferred_element_type=jnp.float32)
    counts = jnp.sum(oh, axis=1, keepdims=True)
    out_ref[0] = sums / jnp.maximum(counts, 1.0)


def _pool(x, labels):
    return pl.pallas_call(
        _pool_body,
        grid=(B,),
        in_specs=[pl.BlockSpec((1, N, C), lambda b: (b, 0, 0)),
                  pl.BlockSpec((1, 1, N), lambda b: (b, 0, 0))],
        out_specs=pl.BlockSpec((1, K, C), lambda b: (b, 0, 0)),
        out_shape=jax.ShapeDtypeStruct((B, K, C), jnp.float32),
    )(x, labels)


@jax.jit
def kernel(x):
    hm = _heatmap(x)
    labels = _cluster_sc(hm.reshape(B, N))
    means = _pool(x, labels.reshape(B, 1, N))
    return tuple(means[:, i, :] for i in range(K))
